# baseline (speedup 1.0000x reference)
import jax
import jax.numpy as jnp
from jax import lax
from jax.experimental import pallas as pl
from jax.experimental.pallas import tpu as pltpu
from jax.experimental.pallas import tpu_sc as plsc

N = 10000
E = 320000
D = 128
CHUNK = 128
CPW = 80
NLANE = 16


def _sc_mesh():
    info = plsc.get_sparse_core_info()
    mesh = plsc.VectorSubcoreMesh(core_axis_name="c", subcore_axis_name="s")
    return info.num_cores, info.num_subcores, mesh


def _row_sliced(s, ns, fn):
    rps = ((N + ns - 1) // ns + 7) // 8 * 8
    last = N - (ns - 1) * rps

    @pl.when(s < ns - 1)
    def _():
        fn(pl.multiple_of(s * rps, 8), rps)

    @pl.when(s == ns - 1)
    def _():
        fn((ns - 1) * rps, last)


def _sc_deg(col2d, ew2d, zeros16):
    nc, ns, mesh = _sc_mesh()

    def body(col_hbm, ew_hbm, z_hbm, out_hbm, acc, colv, ewv, ew16):
        c = lax.axis_index("c")
        s = lax.axis_index("s")
        w = s * nc + c
        _row_sliced(s, ns, lambda o, n: pltpu.sync_copy(
            z_hbm.at[pl.ds(o, n)], acc.at[pl.ds(o, n)]))
        base = pl.multiple_of(w * CPW, 8)
        pltpu.sync_copy(col_hbm.at[pl.ds(base, CPW)], colv)
        pltpu.sync_copy(ew_hbm.at[pl.ds(base, CPW)], ewv)
        plsc.subcore_barrier()

        def chunk_body(i, carry):
            def splat(j, cc):
                ew16[j, :] = plsc.load_gather(
                    ewv, [jnp.full((NLANE,), i, jnp.int32),
                          jnp.full((NLANE,), j, jnp.int32)])
                return cc

            lax.fori_loop(0, CHUNK, splat, 0, unroll=4)
            pltpu.sync_copy(ew16, acc.at[colv.at[i]], add=True)
            return carry

        lax.fori_loop(0, CPW, chunk_body, 0)
        plsc.subcore_barrier()
        _row_sliced(s, ns, lambda o, n: pltpu.sync_copy(
            acc.at[pl.ds(o, n)], out_hbm.at[c, pl.ds(o, n)]))

    return pl.kernel(
        body,
        compiler_params=pltpu.CompilerParams(
            needs_layout_passes=False, use_tc_tiling_on_sc=False),
        out_type=jax.ShapeDtypeStruct((nc, N, NLANE), jnp.float32),
        mesh=mesh,
        scratch_types=[
            pltpu.VMEM_SHARED((N, NLANE), jnp.float32),
            pltpu.VMEM((CPW, CHUNK), jnp.int32),
            pltpu.VMEM((CPW, CHUNK), jnp.float32),
            pltpu.VMEM((CHUNK, NLANE), jnp.float32),
        ],
    )(col2d, ew2d, zeros16)


def _sc_agg(hp, row2d, col2d, ew2d, zeros):
    nc, ns, mesh = _sc_mesh()

    def body(h_hbm, row_hbm, col_hbm, ew_hbm, z_hbm, out_hbm,
             acc, rowv, colv, ewv, buf0, buf1,
             gs0, gs1, ss0, ss1, is0, is1, is2, is3):
        c = lax.axis_index("c")
        s = lax.axis_index("s")
        w = s * nc + c
        isems = (is0, is1, is2, is3)
        bufs = (buf0, buf1)
        gsems = (gs0, gs1)
        ssems = (ss0, ss1)
        _row_sliced(s, ns, lambda o, n: pltpu.sync_copy(
            z_hbm.at[pl.ds(o, n)], acc.at[pl.ds(o, n)]))
        base = pl.multiple_of(w * CPW, 8)

        def fire_idx(i, r):
            pltpu.async_copy(row_hbm.at[base + i], rowv.at[r], isems[r])
            pltpu.async_copy(col_hbm.at[base + i], colv.at[r], isems[r])
            pltpu.async_copy(ew_hbm.at[base + i], ewv.at[r], isems[r])

        def wait_idx(i, r):
            pltpu.make_async_copy(row_hbm.at[base + i], rowv.at[r], isems[r]).wait()
            pltpu.make_async_copy(col_hbm.at[base + i], colv.at[r], isems[r]).wait()
            pltpu.make_async_copy(ew_hbm.at[base + i], ewv.at[r], isems[r]).wait()

        def fire_gather(i, r, b):
            pltpu.async_copy(h_hbm.at[rowv.at[r]], bufs[b], gsems[b])

        def wait_gather(r, b):
            pltpu.make_async_copy(h_hbm.at[rowv.at[r]], bufs[b], gsems[b]).wait()

        def fire_scatter(r, b):
            pltpu.async_copy(bufs[b], acc.at[colv.at[r]], ssems[b], add=True)

        def wait_scatter(r, b):
            pltpu.make_async_copy(bufs[b], acc.at[colv.at[r]], ssems[b]).wait()

        def scale(r, buf):
            def one(j, cc):
                sp = plsc.load_gather(
                    ewv, [jnp.full((NLANE,), r, jnp.int32),
                          jnp.full((NLANE,), j, jnp.int32)])
                for f in range(D // NLANE):
                    sl = pl.ds(f * NLANE, NLANE)
                    buf[j, sl] = buf[j, sl] * sp
                return cc

            lax.fori_loop(0, CHUNK, one, 0, unroll=2)

        plsc.subcore_barrier()
        fire_idx(0, 0)
        fire_idx(1, 1)
        wait_idx(0, 0)
        fire_gather(0, 0, 0)

        def outer(t, carry):
            for b4 in range(4):
                i = 4 * t + b4
                b = b4 % 2
                last_t = CPW // 4 - 1

                def step1():
                    wait_scatter((b4 - 1) % 4, 1 - b)

                if b4 == 0:
                    @pl.when(t > 0)
                    def _():
                        step1()
                else:
                    step1()

                def step2():
                    fire_idx(i + 2, (b4 + 2) % 4)

                if b4 < 2:
                    step2()
                else:
                    @pl.when(t < last_t)
                    def _():
                        step2()

                def step3():
                    wait_idx(i + 1, (b4 + 1) % 4)
                    fire_gather(i + 1, (b4 + 1) % 4, 1 - b)

                if b4 < 3:
                    step3()
                else:
                    @pl.when(t < last_t)
                    def _():
                        step3()

                wait_gather(b4, b)
                scale(b4, bufs[b])
                fire_scatter(b4, b)
            return carry

        lax.fori_loop(0, CPW // 4, outer, 0)
        wait_scatter(3, 1)
        plsc.subcore_barrier()
        _row_sliced(s, ns, lambda o, n: pltpu.sync_copy(
            acc.at[pl.ds(o, n)], out_hbm.at[c, pl.ds(o, n)]))

    return pl.kernel(
        body,
        compiler_params=pltpu.CompilerParams(needs_layout_passes=False),
        out_type=jax.ShapeDtypeStruct((nc, N, D), jnp.float32),
        mesh=mesh,
        scratch_types=[
            pltpu.VMEM_SHARED((N, D), jnp.float32),
            pltpu.VMEM((4, CHUNK), jnp.int32),
            pltpu.VMEM((4, CHUNK), jnp.int32),
            pltpu.VMEM((4, CHUNK), jnp.float32),
            pltpu.VMEM((CHUNK, D), jnp.float32),
            pltpu.VMEM((CHUNK, D), jnp.float32),
            pltpu.SemaphoreType.DMA,
            pltpu.SemaphoreType.DMA,
            pltpu.SemaphoreType.DMA,
            pltpu.SemaphoreType.DMA,
            pltpu.SemaphoreType.DMA,
            pltpu.SemaphoreType.DMA,
            pltpu.SemaphoreType.DMA,
            pltpu.SemaphoreType.DMA,
        ],
    )(hp, row2d, col2d, ew2d, zeros)


def _tc_mm1(x, W, dinv):
    def body(x_ref, w_ref, di_ref, o_ref):
        o_ref[:] = di_ref[:] * jnp.dot(x_ref[:], w_ref[:],
                                       preferred_element_type=jnp.float32)

    return pl.pallas_call(
        body, out_shape=jax.ShapeDtypeStruct((N, D), jnp.float32))(x, W, dinv)


def _tc_dinv(degp):
    def body(d_ref, o_ref):
        deg = d_ref[0, :, 0:1] + d_ref[1, :, 0:1] + 1.0
        o_ref[:] = lax.rsqrt(jnp.maximum(deg, 1e-12))

    return pl.pallas_call(
        body, out_shape=jax.ShapeDtypeStruct((N, 1), jnp.float32))(degp)


def _tc_layer(aggp, hp, x0, dinv, b, g, be, W, relu):
    def body(a_ref, h_ref, x0_ref, di_ref, b_ref, g_ref, be_ref, *rest):
        if W is None:
            (o_ref,) = rest
        else:
            w_ref, o_ref = rest
        di = di_ref[:]
        o = di * (a_ref[0] + a_ref[1] + h_ref[:]) + b_ref[:] + x0_ref[:]
        if relu:
            o = jnp.maximum(o, 0.0)
        mu = jnp.mean(o, axis=0, keepdims=True)
        xc = o - mu
        var = jnp.mean(xc * xc, axis=0, keepdims=True)
        xn = xc * lax.rsqrt(var + 1e-5) * g_ref[:] + be_ref[:]
        if W is None:
            o_ref[:] = xn
        else:
            o_ref[:] = di * jnp.dot(xn, w_ref[:],
                                    preferred_element_type=jnp.float32)

    args = [aggp, hp, x0, dinv, b.reshape(1, D), g.reshape(1, D), be.reshape(1, D)]
    if W is not None:
        args.append(W)
    return pl.pallas_call(
        body, out_shape=jax.ShapeDtypeStruct((N, D), jnp.float32))(*args)


def kernel(x, edge_index, edge_weight, W1, b1, g1, be1, W2, b2, g2, be2,
           W3, b3, g3, be3):
    x = x.astype(jnp.float32)
    ew = edge_weight.astype(jnp.float32)
    nw = 32
    ep = nw * CPW * CHUNK
    pad = ep - E
    row2d = jnp.concatenate(
        [edge_index[0], jnp.zeros((pad,), edge_index.dtype)]).reshape(-1, CHUNK)
    col2d = jnp.concatenate(
        [edge_index[1], jnp.zeros((pad,), edge_index.dtype)]).reshape(-1, CHUNK)
    ew2d = jnp.concatenate([ew, jnp.zeros((pad,), jnp.float32)]).reshape(-1, CHUNK)
    zeros16 = jnp.zeros((N, NLANE), jnp.float32)
    zeros = jnp.zeros((N, D), jnp.float32)

    degp = _sc_deg(col2d, ew2d, zeros16)
    dinv = _tc_dinv(degp)
    hp1 = _tc_mm1(x, W1, dinv)

    agg1 = _sc_agg(hp1, row2d, col2d, ew2d, zeros)
    hp2 = _tc_layer(agg1, hp1, x, dinv, b1, g1, be1, W2, relu=True)
    agg2 = _sc_agg(hp2, row2d, col2d, ew2d, zeros)
    hp3 = _tc_layer(agg2, hp2, x, dinv, b2, g2, be2, W3, relu=True)
    agg3 = _sc_agg(hp3, row2d, col2d, ew2d, zeros)
    out = _tc_layer(agg3, hp3, x, dinv, b3, g3, be3, None, relu=False)
    return out
